# baseline (speedup 1.0000x reference)
import jax
import jax.numpy as jnp
from jax import lax
from jax.experimental import pallas as pl
from jax.experimental.pallas import tpu as pltpu
from jax.experimental.pallas import tpu_sc as plsc

BATCH = 16384
DIM = 64
VOCAB = 1000000
NUM_CORES = 2
NUM_SUBCORES = 16
NUM_WORKERS = NUM_CORES * NUM_SUBCORES
LANES = 16

CHUNK = 128
PANEL_CHUNKS = 8
PANEL_COLS = PANEL_CHUNKS * CHUNK
NPANELS = 31
CAP = 48
NBIN = NPANELS * CAP
DUMMY = DIM * BATCH
EMB_FLAT = DUMMY + CAP * DIM

LAST_FULL_M = 243
TAIL_CHUNK = 7812


def _gather_body(
    labels_hbm, tablet_hbm, tailp_hbm, emb_hbm,
    labs_v, panel_v, binlab_v, binpos_v, counts_v, vals_v, oidx_v, sem,
):
    wid = lax.axis_index("s") * NUM_CORES + lax.axis_index("c")
    iota = lax.iota(jnp.int32, LANES)
    zeros16 = jnp.zeros((LANES,), jnp.int32)

    pltpu.sync_copy(labels_hbm, labs_v)

    for s in range(2):
        counts_v[pl.ds(s * LANES, LANES)] = zeros16

    def init_body(s, carry):
        oidx_v[pl.ds(s * LANES, LANES)] = DUMMY + s * LANES + iota
        return carry

    lax.fori_loop(0, (CAP * DIM) // LANES, init_body, 0)

    def scan_body(i, carry):
        l = labs_v[pl.ds(i * LANES, LANES)]
        pos = i * LANES + iota
        c = jnp.right_shift(l, 7)
        mine = jnp.bitwise_and(c, NUM_WORKERS - 1) == wid
        pan = jnp.right_shift(l, 15)
        cnt, last = plsc.scan_count(pan, mask=mine)
        base = plsc.load_gather(counts_v, [pan], mask=mine)
        bidx = pan * CAP + base + cnt - 1
        plsc.store_scatter(binlab_v, [bidx], l, mask=mine)
        plsc.store_scatter(binpos_v, [bidx], pos, mask=mine)
        plsc.store_scatter(
            counts_v, [pan], base + cnt, mask=jnp.logical_and(mine, last)
        )
        return carry

    lax.fori_loop(0, BATCH // LANES, scan_body, 0)

    for p in range(NPANELS):
        nq = PANEL_CHUNKS if p < NPANELS - 1 else 5
        copies = []
        for q in range(nq):
            m = PANEL_CHUNKS * p + q
            if m <= LAST_FULL_M:
                col = pl.multiple_of((wid + NUM_WORKERS * m) * CHUNK, CHUNK)
                copies.append(
                    pltpu.async_copy(
                        tablet_hbm.at[:, pl.ds(col, CHUNK)],
                        panel_v.at[:, pl.ds(q * CHUNK, CHUNK)],
                        sem,
                    )
                )
            else:
                @pl.when(wid <= 3)
                def _(q=q, m=m):
                    col = pl.multiple_of(
                        (wid + NUM_WORKERS * m) * CHUNK, CHUNK
                    )
                    pltpu.async_copy(
                        tablet_hbm.at[:, pl.ds(col, CHUNK)],
                        panel_v.at[:, pl.ds(q * CHUNK, CHUNK)],
                        sem,
                    ).wait()

                @pl.when(wid == 4)
                def _(q=q):
                    pltpu.async_copy(
                        tailp_hbm,
                        panel_v.at[:, pl.ds(q * CHUNK, CHUNK)],
                        sem,
                    ).wait()

        for cpy in copies:
            cpy.wait()

        c16 = counts_v[pl.ds((p // LANES) * LANES, LANES)]
        n = c16[p % LANES]

        def lab_body(j, carry, p=p):
            bb = jnp.full((LANES,), p * CAP, jnp.int32) + j
            lv = plsc.load_gather(binlab_v, [bb])
            pb = plsc.load_gather(binpos_v, [bb])
            q = jnp.bitwise_and(jnp.right_shift(lv, 12), PANEL_CHUNKS - 1)
            pc = q * CHUNK + jnp.bitwise_and(lv, CHUNK - 1)
            for k in range(DIM // LANES):
                dvec = iota + k * LANES
                vals = plsc.load_gather(panel_v, [dvec, pc])
                off = j * DIM + k * LANES
                vals_v[pl.ds(off, LANES)] = vals
                oidx_v[pl.ds(off, LANES)] = dvec * BATCH + pb
            return carry

        lax.fori_loop(0, n, lab_body, 0)
        pltpu.sync_copy(vals_v, emb_hbm.at[oidx_v])


_sc_gather = pl.kernel(
    _gather_body,
    out_type=jax.ShapeDtypeStruct((EMB_FLAT,), jnp.float32),
    mesh=plsc.VectorSubcoreMesh(
        core_axis_name="c", subcore_axis_name="s", num_cores=NUM_CORES
    ),
    scratch_types=[
        pltpu.VMEM((BATCH,), jnp.int32),
        pltpu.VMEM((DIM, PANEL_COLS), jnp.float32),
        pltpu.VMEM((NBIN,), jnp.int32),
        pltpu.VMEM((NBIN,), jnp.int32),
        pltpu.VMEM((2 * LANES,), jnp.int32),
        pltpu.VMEM((CAP * DIM,), jnp.float32),
        pltpu.VMEM((CAP * DIM,), jnp.int32),
        pltpu.SemaphoreType.DMA,
    ],
    compiler_params=pltpu.CompilerParams(needs_layout_passes=False),
)

BN = 2048


def _mlp_body(embt_ref, w_ref, b_ref, outt_ref):
    h = embt_ref[...]
    h = h * jax.nn.sigmoid(h)
    outt_ref[...] = (
        lax.dot_general(
            w_ref[...], h, (((0,), (0,)), ((), ())),
            preferred_element_type=jnp.float32,
        )
        + b_ref[...]
    )


_tc_mlp = pl.pallas_call(
    _mlp_body,
    grid=(BATCH // BN,),
    in_specs=[
        pl.BlockSpec((DIM, BN), lambda i: (0, i)),
        pl.BlockSpec((DIM, DIM), lambda i: (0, 0)),
        pl.BlockSpec((DIM, 1), lambda i: (0, 0)),
    ],
    out_specs=pl.BlockSpec((DIM, BN), lambda i: (0, i)),
    out_shape=jax.ShapeDtypeStruct((DIM, BATCH), jnp.float32),
)


def kernel(labels, table, W, b):
    tablet = table.T
    tailp = jnp.concatenate(
        [tablet[:, TAIL_CHUNK * CHUNK :], jnp.zeros((DIM, DIM), jnp.float32)],
        axis=1,
    )
    emb_flat = _sc_gather(labels.astype(jnp.int32), tablet, tailp)
    embt = emb_flat[:DUMMY].reshape(DIM, BATCH)
    outt = _tc_mlp(embt, W, b.reshape(DIM, 1))
    return outt.T


# speedup vs baseline: 96.4640x; 96.4640x over previous
import jax
import jax.numpy as jnp
from jax import lax
from jax.experimental import pallas as pl
from jax.experimental.pallas import tpu as pltpu
from jax.experimental.pallas import tpu_sc as plsc

BATCH = 16384
DIM = 64
VOCAB = 1000000
NUM_CORES = 2
NUM_SUBCORES = 16
NUM_WORKERS = NUM_CORES * NUM_SUBCORES
LANES = 16

CHUNK = 128
PANEL_CHUNKS = 8
PANEL_COLS = PANEL_CHUNKS * CHUNK
NPANELS = 31
CAP = 48
NBIN = NPANELS * CAP
ROW = 128
EMB_ROWS = BATCH + NUM_WORKERS * CAP

LAST_FULL_M = 243
TAIL_CHUNK = 7812


def _gather_body(
    labels_hbm, tablet_hbm, tailp_hbm, emb_hbm,
    labs_v, panel_v, binlab_v, binpos_v, counts_v, rowbuf_v, posidx_v, sem,
):
    wid = lax.axis_index("s") * NUM_CORES + lax.axis_index("c")
    iota = lax.iota(jnp.int32, LANES)
    zeros16 = jnp.zeros((LANES,), jnp.int32)

    pltpu.sync_copy(labels_hbm, labs_v)

    for s in range(2):
        counts_v[pl.ds(s * LANES, LANES)] = zeros16

    def scan_body(i, carry):
        l = labs_v[pl.ds(i * LANES, LANES)]
        pos = i * LANES + iota
        c = jnp.right_shift(l, 7)
        mine = jnp.bitwise_and(c, NUM_WORKERS - 1) == wid
        pan = jnp.right_shift(l, 15)
        cnt, last = plsc.scan_count(pan, mask=mine)
        base = plsc.load_gather(counts_v, [pan], mask=mine)
        bidx = pan * CAP + base + cnt - 1
        plsc.store_scatter(binlab_v, [bidx], l, mask=mine)
        plsc.store_scatter(binpos_v, [bidx], pos, mask=mine)
        plsc.store_scatter(
            counts_v, [pan], base + cnt, mask=jnp.logical_and(mine, last)
        )
        return carry

    lax.fori_loop(0, BATCH // LANES, scan_body, 0)

    for p in range(NPANELS):
        nq = PANEL_CHUNKS if p < NPANELS - 1 else 5
        copies = []
        for q in range(nq):
            m = PANEL_CHUNKS * p + q
            if m <= LAST_FULL_M:
                col = pl.multiple_of((wid + NUM_WORKERS * m) * CHUNK, CHUNK)
                copies.append(
                    pltpu.async_copy(
                        tablet_hbm.at[:, pl.ds(col, CHUNK)],
                        panel_v.at[:, pl.ds(q * CHUNK, CHUNK)],
                        sem,
                    )
                )
            else:
                @pl.when(wid <= 3)
                def _(q=q, m=m):
                    col = pl.multiple_of(
                        (wid + NUM_WORKERS * m) * CHUNK, CHUNK
                    )
                    pltpu.async_copy(
                        tablet_hbm.at[:, pl.ds(col, CHUNK)],
                        panel_v.at[:, pl.ds(q * CHUNK, CHUNK)],
                        sem,
                    ).wait()

                @pl.when(wid == 4)
                def _(q=q):
                    pltpu.async_copy(
                        tailp_hbm,
                        panel_v.at[:, pl.ds(q * CHUNK, CHUNK)],
                        sem,
                    ).wait()

        for cpy in copies:
            cpy.wait()

        c16 = counts_v[pl.ds((p // LANES) * LANES, LANES)]
        n = c16[p % LANES]

        for v in range(CAP // LANES):
            posidx_v[pl.ds(v * LANES, LANES)] = (
                BATCH + wid * CAP + v * LANES + iota
            )

        def lab_body(j, carry, p=p):
            bb = jnp.full((LANES,), p * CAP, jnp.int32) + j
            lv = plsc.load_gather(binlab_v, [bb])
            pb = plsc.load_gather(binpos_v, [bb])
            q = jnp.bitwise_and(jnp.right_shift(lv, 12), PANEL_CHUNKS - 1)
            pc = q * CHUNK + jnp.bitwise_and(lv, CHUNK - 1)
            jfull = jnp.full((LANES,), j, jnp.int32)
            for k in range(DIM // LANES):
                dvec = iota + k * LANES
                vals = plsc.load_gather(panel_v, [dvec, pc])
                plsc.store_scatter(rowbuf_v, [jfull, dvec], vals)
            plsc.store_scatter(posidx_v, [jfull], pb, mask=iota == 0)
            return carry

        lax.fori_loop(0, n, lab_body, 0)
        pltpu.sync_copy(rowbuf_v, emb_hbm.at[posidx_v])


_sc_gather = pl.kernel(
    _gather_body,
    out_type=jax.ShapeDtypeStruct((EMB_ROWS, ROW), jnp.float32),
    mesh=plsc.VectorSubcoreMesh(
        core_axis_name="c", subcore_axis_name="s", num_cores=NUM_CORES
    ),
    scratch_types=[
        pltpu.VMEM((BATCH,), jnp.int32),
        pltpu.VMEM((DIM, PANEL_COLS), jnp.float32),
        pltpu.VMEM((NBIN,), jnp.int32),
        pltpu.VMEM((NBIN,), jnp.int32),
        pltpu.VMEM((2 * LANES,), jnp.int32),
        pltpu.VMEM((CAP, ROW), jnp.float32),
        pltpu.VMEM((CAP,), jnp.int32),
        pltpu.SemaphoreType.DMA,
    ],
    compiler_params=pltpu.CompilerParams(needs_layout_passes=False),
)

BN = 2048


def _mlp_body(embp_ref, w_ref, b_ref, outt_ref):
    h = embp_ref[:, :DIM]
    h = h * jax.nn.sigmoid(h)
    outt_ref[...] = (
        lax.dot_general(
            w_ref[...], h, (((0,), (1,)), ((), ())),
            preferred_element_type=jnp.float32,
        )
        + b_ref[...]
    )


_tc_mlp = pl.pallas_call(
    _mlp_body,
    grid=(BATCH // BN,),
    in_specs=[
        pl.BlockSpec((BN, ROW), lambda i: (i, 0)),
        pl.BlockSpec((DIM, DIM), lambda i: (0, 0)),
        pl.BlockSpec((DIM, 1), lambda i: (0, 0)),
    ],
    out_specs=pl.BlockSpec((DIM, BN), lambda i: (0, i)),
    out_shape=jax.ShapeDtypeStruct((DIM, BATCH), jnp.float32),
)


def kernel(labels, table, W, b):
    tablet = table.T
    tailp = jnp.concatenate(
        [tablet[:, TAIL_CHUNK * CHUNK :], jnp.zeros((DIM, DIM), jnp.float32)],
        axis=1,
    )
    emb_pad = _sc_gather(labels.astype(jnp.int32), tablet, tailp)
    outt = _tc_mlp(emb_pad, W, b.reshape(DIM, 1))
    return outt.T


# speedup vs baseline: 118.8137x; 1.2317x over previous
import jax
import jax.numpy as jnp
from jax import lax
from jax.experimental import pallas as pl
from jax.experimental.pallas import tpu as pltpu
from jax.experimental.pallas import tpu_sc as plsc

BATCH = 16384
DIM = 64
VOCAB = 1000000
NUM_CORES = 2
NUM_SUBCORES = 16
NUM_WORKERS = NUM_CORES * NUM_SUBCORES
LANES = 16

CHUNK = 128
PANEL_CHUNKS = 4
PANEL_COLS = PANEL_CHUNKS * CHUNK
NFULL = 61
NPANELS = 62
CAP = 32
NBIN = NPANELS * CAP
ROW = 128
EMB_ROWS = BATCH + NUM_WORKERS * CAP

LAST_FULL_M = 243
TAIL_CHUNK = 7812


def _gather_body(
    labels_hbm, tablet_hbm, tailp_hbm, emb_hbm,
    labs_v, panel_a, panel_b, binlab_v, binpos_v, counts_v,
    rowbuf_a, rowbuf_b, posidx_a, posidx_b,
    dma_a, dma_b, sc_a, sc_b,
):
    wid = lax.axis_index("s") * NUM_CORES + lax.axis_index("c")
    iota = lax.iota(jnp.int32, LANES)
    zeros16 = jnp.zeros((LANES,), jnp.int32)

    pltpu.sync_copy(labels_hbm, labs_v)
    for s in range(4):
        counts_v[pl.ds(s * LANES, LANES)] = zeros16

    def scan_body(i, carry):
        l = labs_v[pl.ds(i * LANES, LANES)]
        pos = i * LANES + iota
        c = jnp.right_shift(l, 7)
        mine = jnp.bitwise_and(c, NUM_WORKERS - 1) == wid
        pan = jnp.right_shift(l, 14)
        cnt, last = plsc.scan_count(pan, mask=mine)
        base = plsc.load_gather(counts_v, [pan], mask=mine)
        bidx = pan * CAP + base + cnt - 1
        plsc.store_scatter(binlab_v, [bidx], l, mask=mine)
        plsc.store_scatter(binpos_v, [bidx], pos, mask=mine)
        plsc.store_scatter(
            counts_v, [pan], base + cnt, mask=jnp.logical_and(mine, last)
        )
        return carry

    lax.fori_loop(0, BATCH // LANES, scan_body, 0)

    def issue_panel(g, buf, sem):
        for q in range(PANEL_CHUNKS):
            col = pl.multiple_of(
                (wid + NUM_WORKERS * (PANEL_CHUNKS * g + q)) * CHUNK, CHUNK
            )
            pltpu.async_copy(
                tablet_hbm.at[:, pl.ds(col, CHUNK)],
                buf.at[:, pl.ds(q * CHUNK, CHUNK)],
                sem,
            )

    def wait_panel(buf, sem):
        pltpu.make_async_copy(
            tablet_hbm.at[:, pl.ds(0, PANEL_COLS)], buf, sem
        ).wait()

    def wait_scatter(rowbuf, posidx, sem):
        pltpu.make_async_copy(rowbuf, emb_hbm.at[posidx], sem).wait()

    def extract(g, buf, rowbuf, posidx, sem):
        nv = plsc.load_gather(counts_v, [jnp.full((LANES,), 0, jnp.int32) + g])
        n = nv[0]
        for v in range(CAP // LANES):
            posidx[pl.ds(v * LANES, LANES)] = (
                BATCH + wid * CAP + v * LANES + iota
            )

        def lab_body(j, carry):
            bb = jnp.full((LANES,), 0, jnp.int32) + (g * CAP + j)
            lv = plsc.load_gather(binlab_v, [bb])
            pb = plsc.load_gather(binpos_v, [bb])
            q = jnp.bitwise_and(jnp.right_shift(lv, 12), PANEL_CHUNKS - 1)
            pc = q * CHUNK + jnp.bitwise_and(lv, CHUNK - 1)
            jfull = jnp.full((LANES,), 0, jnp.int32) + j
            for k in range(DIM // LANES):
                dvec = iota + k * LANES
                vals = plsc.load_gather(buf, [dvec, pc])
                plsc.store_scatter(rowbuf, [jfull, dvec], vals)
            plsc.store_scatter(posidx, [jfull], pb, mask=iota == 0)
            return carry

        lax.fori_loop(0, n, lab_body, 0)
        pltpu.async_copy(rowbuf, emb_hbm.at[posidx], sem)

    issue_panel(0, panel_a, dma_a)
    issue_panel(1, panel_b, dma_b)

    def pipe_body(g2, carry):
        g = 2 * g2
        wait_panel(panel_a, dma_a)

        @pl.when(g2 > 0)
        def _():
            wait_scatter(rowbuf_a, posidx_a, sc_a)

        extract(g, panel_a, rowbuf_a, posidx_a, sc_a)
        issue_panel(g + 2, panel_a, dma_a)

        wait_panel(panel_b, dma_b)

        @pl.when(g2 > 0)
        def _():
            wait_scatter(rowbuf_b, posidx_b, sc_b)

        extract(g + 1, panel_b, rowbuf_b, posidx_b, sc_b)

        @pl.when(g2 < 29)
        def _():
            issue_panel(g + 3, panel_b, dma_b)

        return carry

    lax.fori_loop(0, 30, pipe_body, 0)

    wait_panel(panel_a, dma_a)
    wait_scatter(rowbuf_a, posidx_a, sc_a)
    extract(60, panel_a, rowbuf_a, posidx_a, sc_a)

    wait_scatter(rowbuf_b, posidx_b, sc_b)

    @pl.when(wid <= 3)
    def _():
        col = pl.multiple_of(
            (wid + NUM_WORKERS * (PANEL_CHUNKS * 61)) * CHUNK, CHUNK
        )
        pltpu.async_copy(
            tablet_hbm.at[:, pl.ds(col, CHUNK)],
            panel_b.at[:, pl.ds(0, CHUNK)],
            dma_b,
        ).wait()

    @pl.when(wid == 4)
    def _():
        pltpu.async_copy(
            tailp_hbm, panel_b.at[:, pl.ds(0, CHUNK)], dma_b
        ).wait()

    extract(61, panel_b, rowbuf_b, posidx_b, sc_b)
    wait_scatter(rowbuf_a, posidx_a, sc_a)
    wait_scatter(rowbuf_b, posidx_b, sc_b)


_sc_gather = pl.kernel(
    _gather_body,
    out_type=jax.ShapeDtypeStruct((EMB_ROWS, ROW), jnp.float32),
    mesh=plsc.VectorSubcoreMesh(
        core_axis_name="c", subcore_axis_name="s", num_cores=NUM_CORES
    ),
    scratch_types=[
        pltpu.VMEM((BATCH,), jnp.int32),
        pltpu.VMEM((DIM, PANEL_COLS), jnp.float32),
        pltpu.VMEM((DIM, PANEL_COLS), jnp.float32),
        pltpu.VMEM((NBIN,), jnp.int32),
        pltpu.VMEM((NBIN,), jnp.int32),
        pltpu.VMEM((4 * LANES,), jnp.int32),
        pltpu.VMEM((CAP, ROW), jnp.float32),
        pltpu.VMEM((CAP, ROW), jnp.float32),
        pltpu.VMEM((CAP,), jnp.int32),
        pltpu.VMEM((CAP,), jnp.int32),
        pltpu.SemaphoreType.DMA,
        pltpu.SemaphoreType.DMA,
        pltpu.SemaphoreType.DMA,
        pltpu.SemaphoreType.DMA,
    ],
    compiler_params=pltpu.CompilerParams(needs_layout_passes=False),
)

BN = 2048


def _mlp_body(embp_ref, w_ref, b_ref, outt_ref):
    h = embp_ref[:, :DIM]
    h = h * jax.nn.sigmoid(h)
    outt_ref[...] = (
        lax.dot_general(
            w_ref[...], h, (((0,), (1,)), ((), ())),
            preferred_element_type=jnp.float32,
        )
        + b_ref[...]
    )


_tc_mlp = pl.pallas_call(
    _mlp_body,
    grid=(BATCH // BN,),
    in_specs=[
        pl.BlockSpec((BN, ROW), lambda i: (i, 0)),
        pl.BlockSpec((DIM, DIM), lambda i: (0, 0)),
        pl.BlockSpec((DIM, 1), lambda i: (0, 0)),
    ],
    out_specs=pl.BlockSpec((DIM, BN), lambda i: (0, i)),
    out_shape=jax.ShapeDtypeStruct((DIM, BATCH), jnp.float32),
)


def kernel(labels, table, W, b):
    tablet = table.T
    tailp = jnp.concatenate(
        [tablet[:, TAIL_CHUNK * CHUNK :], jnp.zeros((DIM, DIM), jnp.float32)],
        axis=1,
    )
    emb_pad = _sc_gather(labels.astype(jnp.int32), tablet, tailp)
    outt = _tc_mlp(emb_pad, W, b.reshape(DIM, 1))
    return outt.T


# speedup vs baseline: 123.0402x; 1.0356x over previous
import jax
import jax.numpy as jnp
from jax import lax
from jax.experimental import pallas as pl
from jax.experimental.pallas import tpu as pltpu
from jax.experimental.pallas import tpu_sc as plsc

BATCH = 16384
DIM = 64
VOCAB = 1000000
NUM_CORES = 2
NUM_SUBCORES = 16
NUM_WORKERS = NUM_CORES * NUM_SUBCORES
LANES = 16

CHUNK = 128
PANEL_CHUNKS = 4
PANEL_COLS = PANEL_CHUNKS * CHUNK
NFULL = 61
NPANELS = 62
CAP = 32
NBIN = NPANELS * CAP
ROW = 128
EMB_ROWS = BATCH + NUM_WORKERS * CAP

LAST_FULL_M = 243
TAIL_CHUNK = 7812


def _gather_body(
    labels_hbm, tablet_hbm, tailp_hbm, emb_hbm,
    labs_v, panel_a, panel_b, binlab_v, binpos_v, counts_v, ccnt_v,
    rowbuf_a, rowbuf_b, posidx_a, posidx_b,
    dma_a, dma_b, sc_a, sc_b,
):
    wid = lax.axis_index("s") * NUM_CORES + lax.axis_index("c")
    iota = lax.iota(jnp.int32, LANES)
    zeros16 = jnp.zeros((LANES,), jnp.int32)

    pltpu.sync_copy(labels_hbm, labs_v)
    for s in range(4):
        counts_v[pl.ds(s * LANES, LANES)] = zeros16
    for s in range(16):
        ccnt_v[pl.ds(s * LANES, LANES)] = zeros16

    def scan_body(i, carry):
        l = labs_v[pl.ds(i * LANES, LANES)]
        pos = i * LANES + iota
        c = jnp.right_shift(l, 7)
        mine = jnp.bitwise_and(c, NUM_WORKERS - 1) == wid
        pan = jnp.right_shift(l, 14)
        cnt, last = plsc.scan_count(pan, mask=mine)
        base = plsc.load_gather(counts_v, [pan], mask=mine)
        bidx = pan * CAP + base + cnt - 1
        plsc.store_scatter(binlab_v, [bidx], l, mask=mine)
        plsc.store_scatter(binpos_v, [bidx], pos, mask=mine)
        plsc.store_scatter(
            counts_v, [pan], base + cnt, mask=jnp.logical_and(mine, last)
        )
        m = jnp.right_shift(l, 12)
        cnt2, last2 = plsc.scan_count(m, mask=mine)
        base2 = plsc.load_gather(ccnt_v, [m], mask=mine)
        plsc.store_scatter(
            ccnt_v, [m], base2 + cnt2, mask=jnp.logical_and(mine, last2)
        )
        return carry

    lax.fori_loop(0, BATCH // LANES, scan_body, 0)

    def issue_panel(g, buf, sem):
        for q in range(PANEL_CHUNKS):
            mq = PANEL_CHUNKS * g + q
            nq = plsc.load_gather(
                ccnt_v, [jnp.full((LANES,), 0, jnp.int32) + mq]
            )[0]

            @pl.when(nq > 0)
            def _(q=q, mq=mq):
                col = pl.multiple_of((wid + NUM_WORKERS * mq) * CHUNK, CHUNK)
                pltpu.async_copy(
                    tablet_hbm.at[:, pl.ds(col, CHUNK)],
                    buf.at[:, pl.ds(q * CHUNK, CHUNK)],
                    sem,
                )

    def wait_panel(g, buf, sem):
        for q in range(PANEL_CHUNKS):
            mq = PANEL_CHUNKS * g + q
            nq = plsc.load_gather(
                ccnt_v, [jnp.full((LANES,), 0, jnp.int32) + mq]
            )[0]

            @pl.when(nq > 0)
            def _(q=q):
                pltpu.make_async_copy(
                    tablet_hbm.at[:, pl.ds(0, CHUNK)],
                    buf.at[:, pl.ds(q * CHUNK, CHUNK)],
                    sem,
                ).wait()

    def wait_scatter(rowbuf, posidx, sem):
        pltpu.make_async_copy(rowbuf, emb_hbm.at[posidx], sem).wait()

    def extract(g, buf, rowbuf, posidx, sem):
        nv = plsc.load_gather(counts_v, [jnp.full((LANES,), 0, jnp.int32) + g])
        n = nv[0]
        for v in range(CAP // LANES):
            posidx[pl.ds(v * LANES, LANES)] = (
                BATCH + wid * CAP + v * LANES + iota
            )

        def lab_body(j, carry):
            bb = jnp.full((LANES,), 0, jnp.int32) + (g * CAP + j)
            lv = plsc.load_gather(binlab_v, [bb])
            pb = plsc.load_gather(binpos_v, [bb])
            q = jnp.bitwise_and(jnp.right_shift(lv, 12), PANEL_CHUNKS - 1)
            pc = q * CHUNK + jnp.bitwise_and(lv, CHUNK - 1)
            jfull = jnp.full((LANES,), 0, jnp.int32) + j
            for k in range(DIM // LANES):
                dvec = iota + k * LANES
                vals = plsc.load_gather(buf, [dvec, pc])
                plsc.store_scatter(rowbuf, [jfull, dvec], vals)
            plsc.store_scatter(posidx, [jfull], pb, mask=iota == 0)
            return carry

        lax.fori_loop(0, n, lab_body, 0)
        pltpu.async_copy(rowbuf, emb_hbm.at[posidx], sem)

    issue_panel(0, panel_a, dma_a)
    issue_panel(1, panel_b, dma_b)

    def pipe_body(g2, carry):
        g = 2 * g2
        wait_panel(g, panel_a, dma_a)

        @pl.when(g2 > 0)
        def _():
            wait_scatter(rowbuf_a, posidx_a, sc_a)

        extract(g, panel_a, rowbuf_a, posidx_a, sc_a)
        issue_panel(g + 2, panel_a, dma_a)

        wait_panel(g + 1, panel_b, dma_b)

        @pl.when(g2 > 0)
        def _():
            wait_scatter(rowbuf_b, posidx_b, sc_b)

        extract(g + 1, panel_b, rowbuf_b, posidx_b, sc_b)

        @pl.when(g2 < 29)
        def _():
            issue_panel(g + 3, panel_b, dma_b)

        return carry

    lax.fori_loop(0, 30, pipe_body, 0)

    wait_panel(60, panel_a, dma_a)
    wait_scatter(rowbuf_a, posidx_a, sc_a)
    extract(60, panel_a, rowbuf_a, posidx_a, sc_a)

    wait_scatter(rowbuf_b, posidx_b, sc_b)

    @pl.when(wid <= 3)
    def _():
        col = pl.multiple_of(
            (wid + NUM_WORKERS * (PANEL_CHUNKS * 61)) * CHUNK, CHUNK
        )
        pltpu.async_copy(
            tablet_hbm.at[:, pl.ds(col, CHUNK)],
            panel_b.at[:, pl.ds(0, CHUNK)],
            dma_b,
        ).wait()

    @pl.when(wid == 4)
    def _():
        pltpu.async_copy(
            tailp_hbm, panel_b.at[:, pl.ds(0, CHUNK)], dma_b
        ).wait()

    extract(61, panel_b, rowbuf_b, posidx_b, sc_b)
    wait_scatter(rowbuf_a, posidx_a, sc_a)
    wait_scatter(rowbuf_b, posidx_b, sc_b)


_sc_gather = pl.kernel(
    _gather_body,
    out_type=jax.ShapeDtypeStruct((EMB_ROWS, ROW), jnp.float32),
    mesh=plsc.VectorSubcoreMesh(
        core_axis_name="c", subcore_axis_name="s", num_cores=NUM_CORES
    ),
    scratch_types=[
        pltpu.VMEM((BATCH,), jnp.int32),
        pltpu.VMEM((DIM, PANEL_COLS), jnp.float32),
        pltpu.VMEM((DIM, PANEL_COLS), jnp.float32),
        pltpu.VMEM((NBIN,), jnp.int32),
        pltpu.VMEM((NBIN,), jnp.int32),
        pltpu.VMEM((4 * LANES,), jnp.int32),
        pltpu.VMEM((256,), jnp.int32),
        pltpu.VMEM((CAP, ROW), jnp.float32),
        pltpu.VMEM((CAP, ROW), jnp.float32),
        pltpu.VMEM((CAP,), jnp.int32),
        pltpu.VMEM((CAP,), jnp.int32),
        pltpu.SemaphoreType.DMA,
        pltpu.SemaphoreType.DMA,
        pltpu.SemaphoreType.DMA,
        pltpu.SemaphoreType.DMA,
    ],
    compiler_params=pltpu.CompilerParams(needs_layout_passes=False),
)

BN = 2048


def _mlp_body(embp_ref, w_ref, b_ref, outt_ref):
    h = embp_ref[:, :DIM]
    h = h * jax.nn.sigmoid(h)
    outt_ref[...] = (
        lax.dot_general(
            w_ref[...], h, (((0,), (1,)), ((), ())),
            preferred_element_type=jnp.float32,
        )
        + b_ref[...]
    )


_tc_mlp = pl.pallas_call(
    _mlp_body,
    grid=(BATCH // BN,),
    in_specs=[
        pl.BlockSpec((BN, ROW), lambda i: (i, 0)),
        pl.BlockSpec((DIM, DIM), lambda i: (0, 0)),
        pl.BlockSpec((DIM, 1), lambda i: (0, 0)),
    ],
    out_specs=pl.BlockSpec((DIM, BN), lambda i: (0, i)),
    out_shape=jax.ShapeDtypeStruct((DIM, BATCH), jnp.float32),
)


def kernel(labels, table, W, b):
    tablet = table.T
    tailp = jnp.concatenate(
        [tablet[:, TAIL_CHUNK * CHUNK :], jnp.zeros((DIM, DIM), jnp.float32)],
        axis=1,
    )
    emb_pad = _sc_gather(labels.astype(jnp.int32), tablet, tailp)
    outt = _tc_mlp(emb_pad, W, b.reshape(DIM, 1))
    return outt.T
